# baseline (speedup 1.0000x reference)
import functools

import jax
import jax.numpy as jnp
from jax import lax
from jax.experimental import pallas as pl
from jax.experimental.pallas import tpu as pltpu
from jax.experimental.pallas import tpu_sc as plsc

B = 16
L = 2048
D = 128
NEXT_TOKEN_ID = 5
EPS = 1e-05

N = B * L
NC = 2
NS = 16
NW = NC * NS
PER_W = N // NW
CHUNK = 128
N_CHUNKS = PER_W // CHUNK


def _shift_right(x, s, fill):
    pad = jnp.full((x.shape[0], s), fill, dtype=x.dtype)
    return jnp.concatenate([pad, x[:, : x.shape[1] - s]], axis=1)


def _indices_kernel(state_ref, instr_ref, arg_ref):
    state = state_ref[...]
    eq = (state == NEXT_TOKEN_ID).astype(jnp.int32)
    csum = eq
    s = 1
    while s < L:
        csum = csum + _shift_right(csum, s, 0)
        s *= 2
    instr_ref[...] = csum - eq
    pos = lax.broadcasted_iota(jnp.int32, (B, L), 1)
    m = jnp.where(eq > 0, pos + 1, 0)
    s = 1
    while s < L:
        m = jnp.maximum(m, _shift_right(m, s, 0))
        s *= 2
    seg_start = _shift_right(m, 1, 0)
    arg_ref[...] = pos - seg_start


def _compute_indices(state):
    return pl.pallas_call(
        _indices_kernel,
        out_shape=(
            jax.ShapeDtypeStruct((B, L), jnp.int32),
            jax.ShapeDtypeStruct((B, L), jnp.int32),
        ),
    )(state)


def _gather_sum_kernel(state_hbm, instr_hbm, arg_hbm,
                       tok_tab, ins_tab, arg_tab, out_hbm,
                       idx_tok, idx_ins, idx_arg, acc, sem):
    wid = lax.axis_index("s") * NC + lax.axis_index("c")
    for ci in range(N_CHUNKS):
        base = wid * PER_W + ci * CHUNK
        pltpu.sync_copy(state_hbm.at[pl.ds(base, CHUNK)], idx_tok)
        pltpu.sync_copy(instr_hbm.at[pl.ds(base, CHUNK)], idx_ins)
        pltpu.sync_copy(arg_hbm.at[pl.ds(base, CHUNK)], idx_arg)
        pltpu.async_copy(tok_tab.at[idx_tok], acc, sem).wait()
        pltpu.async_copy(ins_tab.at[idx_ins], acc, sem, add=True).wait()
        pltpu.async_copy(arg_tab.at[idx_arg], acc, sem, add=True).wait()
        pltpu.sync_copy(acc, out_hbm.at[pl.ds(base, CHUNK)])


_gather_sum = functools.partial(
    pl.kernel,
    out_type=jax.ShapeDtypeStruct((N, D), jnp.float32),
    mesh=plsc.VectorSubcoreMesh(core_axis_name="c", subcore_axis_name="s"),
    scratch_types=[
        pltpu.VMEM((CHUNK,), jnp.int32),
        pltpu.VMEM((CHUNK,), jnp.int32),
        pltpu.VMEM((CHUNK,), jnp.int32),
        pltpu.VMEM((CHUNK, D), jnp.float32),
        pltpu.SemaphoreType.DMA,
    ],
)(_gather_sum_kernel)


LN_BLOCK = 1024


def _ln_kernel(x_ref, w_ref, b_ref, o_ref):
    x = x_ref[...]
    mean = jnp.mean(x, axis=-1, keepdims=True)
    d = x - mean
    var = jnp.mean(d * d, axis=-1, keepdims=True)
    rstd = lax.rsqrt(var + EPS)
    o_ref[...] = d * rstd * w_ref[...] + b_ref[...]


def _layernorm(x, w, b):
    return pl.pallas_call(
        _ln_kernel,
        grid=(N // LN_BLOCK,),
        in_specs=[
            pl.BlockSpec((LN_BLOCK, D), lambda i: (i, 0)),
            pl.BlockSpec((1, D), lambda i: (0, 0)),
            pl.BlockSpec((1, D), lambda i: (0, 0)),
        ],
        out_specs=pl.BlockSpec((LN_BLOCK, D), lambda i: (i, 0)),
        out_shape=jax.ShapeDtypeStruct((N, D), jnp.float32),
    )(x, w.reshape(1, D), b.reshape(1, D))


def kernel(state, token_table, instr_table, arg_table, ln_weight, ln_bias):
    instructions, arguments = _compute_indices(state)
    summed = _gather_sum(
        state.reshape(N), instructions.reshape(N), arguments.reshape(N),
        token_table, instr_table, arg_table)
    out = _layernorm(summed, ln_weight, ln_bias)
    return out.reshape(B, L, D)
